# baseline (speedup 1.0000x reference)
import functools
import jax
import jax.numpy as jnp
from jax import lax
from jax.experimental import pallas as pl
from jax.experimental.pallas import tpu as pltpu
from jax.experimental.pallas import tpu_sc as plsc

N, C, H, W = 8, 5, 512, 512
NC, NS, L = 2, 16, 16
NW = NC * NS
TILE_ROWS = 128
BLK_ROWS = 16
NBLK = TILE_ROWS // BLK_ROWS
VECS_PER_ROW = W // L


def _compute_vec(va, vb, vc, vt, accs):
    (a_pi, a_pi2, a_t, a_t2, a_m, a_i1, a_i2) = accs
    zero = jnp.zeros((L,), jnp.int32)
    one = jnp.full((L,), 1, jnp.int32)
    two = jnp.full((L,), 2, jnp.int32)
    gtb = vb > va
    mx = jnp.maximum(va, vb)
    gtc = vc > mx
    idx = jnp.where(gtc, two, jnp.where(gtb, one, zero))
    meq = idx == vt
    ie = jnp.where(meq, vt, zero)
    a_pi = a_pi + idx
    a_pi2 = a_pi2 + idx * idx
    a_t = a_t + vt
    a_t2 = a_t2 + vt * vt
    a_m = a_m + jnp.where(meq, one, zero)
    a_i1 = a_i1 + ie
    a_i2 = a_i2 + ie * vt
    return (a_pi, a_pi2, a_t, a_t2, a_m, a_i1, a_i2)


def _sc_partials(output, target):
    mesh = plsc.VectorSubcoreMesh(core_axis_name="c", subcore_axis_name="s")

    @functools.partial(
        pl.kernel,
        mesh=mesh,
        out_type=jax.ShapeDtypeStruct((NW, 8, L), jnp.int32),
        scratch_types=[
            pltpu.VMEM((BLK_ROWS, W), jnp.float32),
            pltpu.VMEM((BLK_ROWS, W), jnp.float32),
            pltpu.VMEM((BLK_ROWS, W), jnp.float32),
            pltpu.VMEM((BLK_ROWS, W), jnp.int32),
            pltpu.VMEM((8, L), jnp.int32),
        ],
    )
    def k(out_hbm, tgt_hbm, res_hbm, av, bv, cv, tv, ov):
        wid = lax.axis_index("s") * NC + lax.axis_index("c")
        n = wid // 4
        row0 = (wid % 4) * TILE_ROWS

        zero = jnp.zeros((L,), jnp.int32)
        accs = (zero, zero, zero, zero, zero, zero, zero)

        def block(i, accs):
            r0 = row0 + i * BLK_ROWS
            pltpu.sync_copy(out_hbm.at[n, 2, pl.ds(r0, BLK_ROWS)], av)
            pltpu.sync_copy(out_hbm.at[n, 3, pl.ds(r0, BLK_ROWS)], bv)
            pltpu.sync_copy(out_hbm.at[n, 4, pl.ds(r0, BLK_ROWS)], cv)
            pltpu.sync_copy(tgt_hbm.at[n, 2, pl.ds(r0, BLK_ROWS)], tv)

            def body(j, accs):
                c0 = j * L
                for r in range(BLK_ROWS):
                    va = av[r, pl.ds(c0, L)]
                    vb = bv[r, pl.ds(c0, L)]
                    vc = cv[r, pl.ds(c0, L)]
                    vt = tv[r, pl.ds(c0, L)]
                    accs = _compute_vec(va, vb, vc, vt, accs)
                return accs

            return lax.fori_loop(0, VECS_PER_ROW, body, accs)

        accs = lax.fori_loop(0, NBLK, block, accs)

        for idx8 in range(7):
            ov[idx8, :] = accs[idx8]
        ov[7, :] = zero
        pltpu.sync_copy(ov, res_hbm.at[wid])

    return k(output, target)


def kernel(output, target):
    tgt = target.astype(jnp.int32)
    parts = _sc_partials(output, tgt)
    s = parts.astype(jnp.float32).sum(axis=2)
    per = s.reshape(N, 4, 8).sum(axis=1)
    pi, pi2 = per[:, 0], per[:, 1]
    tt, tt2 = per[:, 2], per[:, 3]
    mq, i1e, i2e = per[:, 4], per[:, 5], per[:, 6]
    m = jnp.float32(H * W)
    p2 = (pi2 - pi) * 0.5
    p1 = pi - 2.0 * p2
    p0 = m - p1 - p2
    t2c = (tt2 - tt) * 0.5
    t1c = tt - 2.0 * t2c
    t0c = m - t1c - t2c
    i2c = (i2e - i1e) * 0.5
    i1c = i1e - 2.0 * i2c
    i0c = mq - i1c - i2c
    eps = jnp.float32(1e-10)
    d0 = 2.0 * i0c / (p0 + t0c + eps)
    d1 = 2.0 * i1c / (p1 + t1c + eps)
    d2 = 2.0 * i2c / (p2 + t2c + eps)
    return jnp.stack([jnp.mean(d0), jnp.mean(d1), jnp.mean(d2)])


# speedup vs baseline: 2.0375x; 2.0375x over previous
import functools
import jax
import jax.numpy as jnp
from jax import lax
from jax.experimental import pallas as pl
from jax.experimental.pallas import tpu as pltpu
from jax.experimental.pallas import tpu_sc as plsc

N, C, H, W = 8, 5, 512, 512
NC, NS, L = 2, 16, 16
NW = NC * NS
TILE_ROWS = 128
BLK_ROWS = 16
NBLK = TILE_ROWS // BLK_ROWS
VECS_PER_ROW = W // L


def _compute_vec(va, vb, vc, vt, accs, consts):
    a_p, a_t, a_i, a_m = accs
    zero, one, k1, k2 = consts
    gtb = vb > va
    mx = jnp.maximum(va, vb)
    gtc = vc > mx
    pp = jnp.where(gtc, k2, jnp.where(gtb, k1, zero))
    pt = vt + ((vt * vt) << 16)
    meq = pp == pt
    a_p = a_p + pp
    a_t = a_t + pt
    a_i = a_i + jnp.where(meq, pt, zero)
    a_m = a_m + jnp.where(meq, one, zero)
    return (a_p, a_t, a_i, a_m)


def _sc_partials(output, target):
    mesh = plsc.VectorSubcoreMesh(core_axis_name="c", subcore_axis_name="s")

    @functools.partial(
        pl.kernel,
        mesh=mesh,
        out_type=jax.ShapeDtypeStruct((NW, 4, L), jnp.int32),
        scratch_types=[
            pltpu.VMEM((2, BLK_ROWS, W), jnp.float32),
            pltpu.VMEM((2, BLK_ROWS, W), jnp.float32),
            pltpu.VMEM((2, BLK_ROWS, W), jnp.float32),
            pltpu.VMEM((2, BLK_ROWS, W), jnp.int32),
            pltpu.VMEM((4, L), jnp.int32),
            pltpu.SemaphoreType.DMA,
            pltpu.SemaphoreType.DMA,
        ],
    )
    def k(out_hbm, tgt_hbm, res_hbm, av, bv, cv, tv, ov, sem0, sem1):
        wid = lax.axis_index("s") * NC + lax.axis_index("c")
        n = wid // 4
        row0 = (wid % 4) * TILE_ROWS
        sems = (sem0, sem1)

        def issue(i):
            par = i % 2
            r0 = row0 + i * BLK_ROWS
            s = sems[par]
            return [
                pltpu.async_copy(out_hbm.at[n, 2, pl.ds(r0, BLK_ROWS)],
                                 av.at[par], s),
                pltpu.async_copy(out_hbm.at[n, 3, pl.ds(r0, BLK_ROWS)],
                                 bv.at[par], s),
                pltpu.async_copy(out_hbm.at[n, 4, pl.ds(r0, BLK_ROWS)],
                                 cv.at[par], s),
                pltpu.async_copy(tgt_hbm.at[n, 2, pl.ds(r0, BLK_ROWS)],
                                 tv.at[par], s),
            ]

        zero = jnp.zeros((L,), jnp.int32)
        one = jnp.full((L,), 1, jnp.int32)
        k1 = jnp.full((L,), 1 + (1 << 16), jnp.int32)
        k2 = jnp.full((L,), 2 + (4 << 16), jnp.int32)
        consts = (zero, one, k1, k2)

        accs = (zero, zero, zero, zero)
        handles = issue(0)
        for i in range(NBLK):
            nxt = issue(i + 1) if i + 1 < NBLK else None
            for h in handles:
                h.wait()
            handles = nxt
            par = i % 2

            def body(j, accs, par=par):
                c0 = j * L
                for r in range(BLK_ROWS):
                    va = av[par, r, pl.ds(c0, L)]
                    vb = bv[par, r, pl.ds(c0, L)]
                    vc = cv[par, r, pl.ds(c0, L)]
                    vt = tv[par, r, pl.ds(c0, L)]
                    accs = _compute_vec(va, vb, vc, vt, accs, consts)
                return accs

            accs = lax.fori_loop(0, VECS_PER_ROW, body, accs)

        for row in range(4):
            ov[row, :] = accs[row]
        pltpu.sync_copy(ov, res_hbm.at[wid])

    return k(output, target)


def kernel(output, target):
    tgt = target.astype(jnp.int32)
    parts = _sc_partials(output, tgt)
    lo = (parts & 0xFFFF).astype(jnp.float32)
    hi = (parts >> 16).astype(jnp.float32)
    lo = lo.sum(axis=2).reshape(N, 4, 4).sum(axis=1)
    hi = hi.sum(axis=2).reshape(N, 4, 4).sum(axis=1)
    pi, pi2 = lo[:, 0], hi[:, 0]
    tt, tt2 = lo[:, 1], hi[:, 1]
    i1e, i2e = lo[:, 2], hi[:, 2]
    mq = lo[:, 3]
    m = jnp.float32(H * W)
    p2 = (pi2 - pi) * 0.5
    p1 = pi - 2.0 * p2
    p0 = m - p1 - p2
    t2c = (tt2 - tt) * 0.5
    t1c = tt - 2.0 * t2c
    t0c = m - t1c - t2c
    i2c = (i2e - i1e) * 0.5
    i1c = i1e - 2.0 * i2c
    i0c = mq - i1c - i2c
    eps = jnp.float32(1e-10)
    d0 = 2.0 * i0c / (p0 + t0c + eps)
    d1 = 2.0 * i1c / (p1 + t1c + eps)
    d2 = 2.0 * i2c / (p2 + t2c + eps)
    return jnp.stack([jnp.mean(d0), jnp.mean(d1), jnp.mean(d2)])


# speedup vs baseline: 2.0446x; 1.0034x over previous
import functools
import jax
import jax.numpy as jnp
from jax import lax
from jax.experimental import pallas as pl
from jax.experimental.pallas import tpu as pltpu
from jax.experimental.pallas import tpu_sc as plsc

N, C, H, W = 8, 5, 512, 512
NC, NS, L = 2, 16, 16
NW = NC * NS

SC_ROWS = 128
TILE_ROWS = SC_ROWS // 4
BLK_ROWS = 16
NBLK = TILE_ROWS // BLK_ROWS
VECS_PER_ROW = W // L

TC_ROWS = H - SC_ROWS
TC_BR = 128
TC_NB = TC_ROWS // TC_BR


def _compute_vec(va, vb, vc, vt, accs, consts):
    a_p, a_t, a_i, a_m = accs
    zero, one, k1, k2 = consts
    gtb = vb > va
    mx = jnp.maximum(va, vb)
    gtc = vc > mx
    pp = jnp.where(gtc, k2, jnp.where(gtb, k1, zero))
    pt = vt + ((vt * vt) << 16)
    meq = pp == pt
    a_p = a_p + pp
    a_t = a_t + pt
    a_i = a_i + jnp.where(meq, pt, zero)
    a_m = a_m + jnp.where(meq, one, zero)
    return (a_p, a_t, a_i, a_m)


def _sc_partials(output, target):
    mesh = plsc.VectorSubcoreMesh(core_axis_name="c", subcore_axis_name="s")

    @functools.partial(
        pl.kernel,
        mesh=mesh,
        out_type=jax.ShapeDtypeStruct((NW, 4, L), jnp.int32),
        scratch_types=[
            pltpu.VMEM((2, BLK_ROWS, W), jnp.float32),
            pltpu.VMEM((2, BLK_ROWS, W), jnp.float32),
            pltpu.VMEM((2, BLK_ROWS, W), jnp.float32),
            pltpu.VMEM((2, BLK_ROWS, W), jnp.int32),
            pltpu.VMEM((4, L), jnp.int32),
            pltpu.SemaphoreType.DMA,
            pltpu.SemaphoreType.DMA,
        ],
    )
    def k(out_hbm, tgt_hbm, res_hbm, av, bv, cv, tv, ov, sem0, sem1):
        wid = lax.axis_index("s") * NC + lax.axis_index("c")
        n = wid // 4
        row0 = (wid % 4) * TILE_ROWS
        sems = (sem0, sem1)

        def issue(i):
            par = i % 2
            r0 = row0 + i * BLK_ROWS
            s = sems[par]
            return [
                pltpu.async_copy(out_hbm.at[n, 2, pl.ds(r0, BLK_ROWS)],
                                 av.at[par], s),
                pltpu.async_copy(out_hbm.at[n, 3, pl.ds(r0, BLK_ROWS)],
                                 bv.at[par], s),
                pltpu.async_copy(out_hbm.at[n, 4, pl.ds(r0, BLK_ROWS)],
                                 cv.at[par], s),
                pltpu.async_copy(tgt_hbm.at[n, 2, pl.ds(r0, BLK_ROWS)],
                                 tv.at[par], s),
            ]

        zero = jnp.zeros((L,), jnp.int32)
        one = jnp.full((L,), 1, jnp.int32)
        k1 = jnp.full((L,), 1 + (1 << 16), jnp.int32)
        k2 = jnp.full((L,), 2 + (4 << 16), jnp.int32)
        consts = (zero, one, k1, k2)

        accs = (zero, zero, zero, zero)
        handles = issue(0)
        for i in range(NBLK):
            nxt = issue(i + 1) if i + 1 < NBLK else None
            for h in handles:
                h.wait()
            handles = nxt
            par = i % 2

            def body(j, accs, par=par):
                c0 = j * L
                for r in range(BLK_ROWS):
                    va = av[par, r, pl.ds(c0, L)]
                    vb = bv[par, r, pl.ds(c0, L)]
                    vc = cv[par, r, pl.ds(c0, L)]
                    vt = tv[par, r, pl.ds(c0, L)]
                    accs = _compute_vec(va, vb, vc, vt, accs, consts)
                return accs

            accs = lax.fori_loop(0, VECS_PER_ROW, body, accs)

        for row in range(4):
            ov[row, :] = accs[row]
        pltpu.sync_copy(ov, res_hbm.at[wid])

    return k(output, target)


def _tc_body(a_ref, b_ref, c_ref, t_ref, o_ref):
    j = pl.program_id(1)

    @pl.when(j == 0)
    def _():
        o_ref[...] = jnp.zeros_like(o_ref)

    a = a_ref[0, 0]
    b = b_ref[0, 0]
    c = c_ref[0, 0]
    t = t_ref[0, 0]
    gtb = b > a
    gtc = c > jnp.maximum(a, b)
    p2m = gtc
    p1m = jnp.logical_and(gtb, jnp.logical_not(gtc))
    p0m = jnp.logical_not(jnp.logical_or(gtb, gtc))
    t1m = t == 1
    t2m = t == 2
    t0m = jnp.logical_not(jnp.logical_or(t1m, t2m))
    stats = (
        p1m, p2m, t1m, t2m,
        jnp.logical_and(p0m, t0m),
        jnp.logical_and(p1m, t1m),
        jnp.logical_and(p2m, t2m),
    )
    for s, msk in enumerate(stats):
        v = jnp.where(msk, 1.0, 0.0).reshape(TC_BR // 8, 8, W).sum(axis=0)
        o_ref[0, s] += v


def _tc_partials(output, target):
    roff = SC_ROWS // TC_BR
    return pl.pallas_call(
        _tc_body,
        grid=(N, TC_NB),
        in_specs=[
            pl.BlockSpec((1, 1, TC_BR, W), lambda n, j: (n, 2, roff + j, 0)),
            pl.BlockSpec((1, 1, TC_BR, W), lambda n, j: (n, 3, roff + j, 0)),
            pl.BlockSpec((1, 1, TC_BR, W), lambda n, j: (n, 4, roff + j, 0)),
            pl.BlockSpec((1, 1, TC_BR, W), lambda n, j: (n, 2, roff + j, 0)),
        ],
        out_specs=pl.BlockSpec((1, 7, 8, W), lambda n, j: (n, 0, 0, 0)),
        out_shape=jax.ShapeDtypeStruct((N, 7, 8, W), jnp.float32),
        compiler_params=pltpu.CompilerParams(
            dimension_semantics=("parallel", "arbitrary"),
        ),
    )(output, output, output, target)


def kernel(output, target):
    tgt = target.astype(jnp.int32)
    sc_parts = _sc_partials(output, tgt)
    tc_parts = _tc_partials(output, tgt)

    lo = (sc_parts & 0xFFFF).astype(jnp.float32)
    hi = (sc_parts >> 16).astype(jnp.float32)
    lo = lo.sum(axis=2).reshape(N, 4, 4).sum(axis=1)
    hi = hi.sum(axis=2).reshape(N, 4, 4).sum(axis=1)
    pi, pi2 = lo[:, 0], hi[:, 0]
    tt, tt2 = lo[:, 1], hi[:, 1]
    i1e, i2e = lo[:, 2], hi[:, 2]
    mq = lo[:, 3]
    p2s = (pi2 - pi) * 0.5
    p1s = pi - 2.0 * p2s
    t2s = (tt2 - tt) * 0.5
    t1s = tt - 2.0 * t2s
    i2s = (i2e - i1e) * 0.5
    i1s = i1e - 2.0 * i2s
    i0s = mq - i1s - i2s

    tc = tc_parts.sum(axis=(2, 3))
    p1 = p1s + tc[:, 0]
    p2 = p2s + tc[:, 1]
    t1c = t1s + tc[:, 2]
    t2c = t2s + tc[:, 3]
    i0c = i0s + tc[:, 4]
    i1c = i1s + tc[:, 5]
    i2c = i2s + tc[:, 6]
    m = jnp.float32(H * W)
    p0 = m - p1 - p2
    t0c = m - t1c - t2c
    eps = jnp.float32(1e-10)
    d0 = 2.0 * i0c / (p0 + t0c + eps)
    d1 = 2.0 * i1c / (p1 + t1c + eps)
    d2 = 2.0 * i2c / (p2 + t2c + eps)
    return jnp.stack([jnp.mean(d0), jnp.mean(d1), jnp.mean(d2)])


# speedup vs baseline: 2.3555x; 1.1521x over previous
import functools
import jax
import jax.numpy as jnp
from jax import lax
from jax.experimental import pallas as pl
from jax.experimental.pallas import tpu as pltpu
from jax.experimental.pallas import tpu_sc as plsc

N, C, H, W = 8, 5, 512, 512
NC, NS, L = 2, 16, 16
NW = NC * NS

TC_ROWS = 384
TC_BR = 128
TC_NB = TC_ROWS // TC_BR

SC_ROWS = H - TC_ROWS
TILE_ROWS = SC_ROWS // 4
BLK_ROWS = 16
NBLK = TILE_ROWS // BLK_ROWS
VECS_PER_ROW = W // L

_K1 = 1 + (1 << 16)
_K2 = 2 + (4 << 16)


def _compute_vec(va, vb, vc, vt, accs, consts):
    a_p, a_t, a_i, a_m = accs
    zero, one, k1, k2 = consts
    gtb = vb > va
    mx = jnp.maximum(va, vb)
    gtc = vc > mx
    pp = jnp.where(gtc, k2, jnp.where(gtb, k1, zero))
    pt = vt + ((vt * vt) << 16)
    meq = pp == pt
    a_p = a_p + pp
    a_t = a_t + pt
    a_i = a_i + jnp.where(meq, pt, zero)
    a_m = a_m + jnp.where(meq, one, zero)
    return (a_p, a_t, a_i, a_m)


def _sc_partials(output, target):
    mesh = plsc.VectorSubcoreMesh(core_axis_name="c", subcore_axis_name="s")

    @functools.partial(
        pl.kernel,
        mesh=mesh,
        out_type=jax.ShapeDtypeStruct((NW, 4, L), jnp.int32),
        scratch_types=[
            pltpu.VMEM((2, BLK_ROWS, W), jnp.float32),
            pltpu.VMEM((2, BLK_ROWS, W), jnp.float32),
            pltpu.VMEM((2, BLK_ROWS, W), jnp.float32),
            pltpu.VMEM((2, BLK_ROWS, W), jnp.int32),
            pltpu.VMEM((4, L), jnp.int32),
            pltpu.SemaphoreType.DMA,
            pltpu.SemaphoreType.DMA,
        ],
    )
    def k(out_hbm, tgt_hbm, res_hbm, av, bv, cv, tv, ov, sem0, sem1):
        wid = lax.axis_index("s") * NC + lax.axis_index("c")
        n = wid // 4
        row0 = TC_ROWS + (wid % 4) * TILE_ROWS
        sems = (sem0, sem1)

        def issue(i):
            par = i % 2
            r0 = row0 + i * BLK_ROWS
            s = sems[par]
            return [
                pltpu.async_copy(out_hbm.at[n, 2, pl.ds(r0, BLK_ROWS)],
                                 av.at[par], s),
                pltpu.async_copy(out_hbm.at[n, 3, pl.ds(r0, BLK_ROWS)],
                                 bv.at[par], s),
                pltpu.async_copy(out_hbm.at[n, 4, pl.ds(r0, BLK_ROWS)],
                                 cv.at[par], s),
                pltpu.async_copy(tgt_hbm.at[n, 2, pl.ds(r0, BLK_ROWS)],
                                 tv.at[par], s),
            ]

        zero = jnp.zeros((L,), jnp.int32)
        one = jnp.full((L,), 1, jnp.int32)
        k1 = jnp.full((L,), _K1, jnp.int32)
        k2 = jnp.full((L,), _K2, jnp.int32)
        consts = (zero, one, k1, k2)

        accs = (zero, zero, zero, zero)
        handles = issue(0)
        for i in range(NBLK):
            nxt = issue(i + 1) if i + 1 < NBLK else None
            for h in handles:
                h.wait()
            handles = nxt
            par = i % 2

            def body(j, accs, par=par):
                c0 = j * L
                for r in range(BLK_ROWS):
                    va = av[par, r, pl.ds(c0, L)]
                    vb = bv[par, r, pl.ds(c0, L)]
                    vc = cv[par, r, pl.ds(c0, L)]
                    vt = tv[par, r, pl.ds(c0, L)]
                    accs = _compute_vec(va, vb, vc, vt, accs, consts)
                return accs

            accs = lax.fori_loop(0, VECS_PER_ROW, body, accs)

        for row in range(4):
            ov[row, :] = accs[row]
        pltpu.sync_copy(ov, res_hbm.at[wid])

    return k(output, target)


def _tc_body(a_ref, b_ref, c_ref, t_ref, o_ref):
    j = pl.program_id(1)

    @pl.when(j == 0)
    def _():
        o_ref[...] = jnp.zeros_like(o_ref)

    a = a_ref[0, 0]
    b = b_ref[0, 0]
    c = c_ref[0, 0]
    t = t_ref[0, 0]
    gtb = b > a
    gtc = c > jnp.maximum(a, b)
    zero = jnp.int32(0)
    pp = jnp.where(gtc, jnp.int32(_K2), jnp.where(gtb, jnp.int32(_K1), zero))
    pt = t + ((t * t) << 16)
    meq = pp == pt
    vals = (pp, pt, jnp.where(meq, pt, zero), jnp.where(meq, jnp.int32(1), zero))
    for s, v in enumerate(vals):
        o_ref[0, s] += v.reshape(TC_BR // 8, 8, W).sum(axis=0)


def _tc_partials(output, target):
    return pl.pallas_call(
        _tc_body,
        grid=(N, TC_NB),
        in_specs=[
            pl.BlockSpec((1, 1, TC_BR, W), lambda n, j: (n, 2, j, 0)),
            pl.BlockSpec((1, 1, TC_BR, W), lambda n, j: (n, 3, j, 0)),
            pl.BlockSpec((1, 1, TC_BR, W), lambda n, j: (n, 4, j, 0)),
            pl.BlockSpec((1, 1, TC_BR, W), lambda n, j: (n, 2, j, 0)),
        ],
        out_specs=pl.BlockSpec((1, 4, 8, W), lambda n, j: (n, 0, 0, 0)),
        out_shape=jax.ShapeDtypeStruct((N, 4, 8, W), jnp.int32),
        compiler_params=pltpu.CompilerParams(
            dimension_semantics=("parallel", "arbitrary"),
        ),
    )(output, output, output, target)


def _combine_body(sc_ref, tc_ref, o_ref):
    sc = sc_ref[...]
    slo = (sc & 0xFFFF).astype(jnp.float32)
    shi = (sc >> 16).astype(jnp.float32)
    slo = slo.sum(axis=2).reshape(N, 4, 4).sum(axis=1)
    shi = shi.sum(axis=2).reshape(N, 4, 4).sum(axis=1)
    tc = tc_ref[...]
    tlo = (tc & 0xFFFF).astype(jnp.float32).sum(axis=(2, 3))
    thi = (tc >> 16).astype(jnp.float32).sum(axis=(2, 3))
    lo = slo + tlo
    hi = shi + thi
    pi, pi2 = lo[:, 0], hi[:, 0]
    tt, tt2 = lo[:, 1], hi[:, 1]
    i1e, i2e = lo[:, 2], hi[:, 2]
    mq = lo[:, 3]
    m = jnp.float32(H * W)
    p2 = (pi2 - pi) * 0.5
    p1 = pi - 2.0 * p2
    p0 = m - p1 - p2
    t2c = (tt2 - tt) * 0.5
    t1c = tt - 2.0 * t2c
    t0c = m - t1c - t2c
    i2c = (i2e - i1e) * 0.5
    i1c = i1e - 2.0 * i2c
    i0c = mq - i1c - i2c
    eps = jnp.float32(1e-10)
    d0 = jnp.mean(2.0 * i0c / (p0 + t0c + eps))
    d1 = jnp.mean(2.0 * i1c / (p1 + t1c + eps))
    d2 = jnp.mean(2.0 * i2c / (p2 + t2c + eps))
    li = lax.broadcasted_iota(jnp.int32, (8, 128), 1)
    out = jnp.where(li == 0, d0, jnp.where(li == 1, d1,
                    jnp.where(li == 2, d2, 0.0)))
    o_ref[...] = out


def _combine(sc_parts, tc_parts):
    return pl.pallas_call(
        _combine_body,
        out_shape=jax.ShapeDtypeStruct((8, 128), jnp.float32),
    )(sc_parts, tc_parts)


def kernel(output, target):
    tgt = target.astype(jnp.int32)
    sc_parts = _sc_partials(output, tgt)
    tc_parts = _tc_partials(output, tgt)
    res = _combine(sc_parts, tc_parts)
    return res[0, :3]


# speedup vs baseline: 2.5632x; 1.0882x over previous
import functools
import jax
import jax.numpy as jnp
from jax import lax
from jax.experimental import pallas as pl
from jax.experimental.pallas import tpu as pltpu
from jax.experimental.pallas import tpu_sc as plsc

N, C, H, W = 8, 5, 512, 512
NC, NS, L = 2, 16, 16
NW = NC * NS

TC_ROWS = 320

SC_ROWS = H - TC_ROWS
TILE_ROWS = SC_ROWS // 4
BLK_ROWS = 16
NBLK = TILE_ROWS // BLK_ROWS
VECS_PER_ROW = W // L

_K1 = 1 + (1 << 16)
_K2 = 2 + (4 << 16)


def _compute_vec(va, vb, vc, vt, accs, consts):
    a_p, a_t, a_i, a_m = accs
    zero, one, k1, k2 = consts
    gtb = vb > va
    mx = jnp.maximum(va, vb)
    gtc = vc > mx
    pp = jnp.where(gtc, k2, jnp.where(gtb, k1, zero))
    pt = vt + ((vt * vt) << 16)
    meq = pp == pt
    a_p = a_p + pp
    a_t = a_t + pt
    a_i = a_i + jnp.where(meq, pt, zero)
    a_m = a_m + jnp.where(meq, one, zero)
    return (a_p, a_t, a_i, a_m)


def _sc_partials(output, target):
    mesh = plsc.VectorSubcoreMesh(core_axis_name="c", subcore_axis_name="s")

    @functools.partial(
        pl.kernel,
        mesh=mesh,
        out_type=jax.ShapeDtypeStruct((NW, 4, L), jnp.int32),
        scratch_types=[
            pltpu.VMEM((2, 3, BLK_ROWS, W), jnp.float32),
            pltpu.VMEM((2, BLK_ROWS, W), jnp.int32),
            pltpu.VMEM((4, L), jnp.int32),
            pltpu.SemaphoreType.DMA,
            pltpu.SemaphoreType.DMA,
        ],
    )
    def k(out_hbm, tgt_hbm, res_hbm, xv, tv, ov, sem0, sem1):
        wid = lax.axis_index("s") * NC + lax.axis_index("c")
        n = wid // 4
        row0 = TC_ROWS + (wid % 4) * TILE_ROWS
        sems = (sem0, sem1)

        def issue(i):
            par = i % 2
            r0 = row0 + i * BLK_ROWS
            s = sems[par]
            return [
                pltpu.async_copy(
                    out_hbm.at[n, pl.ds(2, 3), pl.ds(r0, BLK_ROWS)],
                    xv.at[par], s),
                pltpu.async_copy(tgt_hbm.at[n, 2, pl.ds(r0, BLK_ROWS)],
                                 tv.at[par], s),
            ]

        zero = jnp.zeros((L,), jnp.int32)
        one = jnp.full((L,), 1, jnp.int32)
        k1 = jnp.full((L,), _K1, jnp.int32)
        k2 = jnp.full((L,), _K2, jnp.int32)
        consts = (zero, one, k1, k2)

        accs = (zero, zero, zero, zero)
        handles = issue(0)
        for i in range(NBLK):
            nxt = issue(i + 1) if i + 1 < NBLK else None
            for h in handles:
                h.wait()
            handles = nxt
            par = i % 2

            def body(j, accs, par=par):
                c0 = j * L
                for r in range(BLK_ROWS):
                    va = xv[par, 0, r, pl.ds(c0, L)]
                    vb = xv[par, 1, r, pl.ds(c0, L)]
                    vc = xv[par, 2, r, pl.ds(c0, L)]
                    vt = tv[par, r, pl.ds(c0, L)]
                    accs = _compute_vec(va, vb, vc, vt, accs, consts)
                return accs

            accs = lax.fori_loop(0, VECS_PER_ROW, body, accs)

        for row in range(4):
            ov[row, :] = accs[row]
        pltpu.sync_copy(ov, res_hbm.at[wid])

    return k(output, target)


def _tc_body(a_ref, b_ref, c_ref, t_ref, o_ref):
    a = a_ref[0, 0]
    b = b_ref[0, 0]
    c = c_ref[0, 0]
    t = t_ref[0, 0]
    gtb = b > a
    gtc = c > jnp.maximum(a, b)
    zero = jnp.int32(0)
    pp = jnp.where(gtc, jnp.int32(_K2), jnp.where(gtb, jnp.int32(_K1), zero))
    pt = t + ((t * t) << 16)
    meq = pp == pt
    vals = (pp, pt, jnp.where(meq, pt, zero), jnp.where(meq, jnp.int32(1), zero))
    for s, v in enumerate(vals):
        o_ref[0, s] = v.reshape(TC_ROWS // 8, 8, 4, 128).sum(axis=(0, 2))


def _tc_partials(output, target):
    return pl.pallas_call(
        _tc_body,
        grid=(N,),
        in_specs=[
            pl.BlockSpec((1, 1, TC_ROWS, W), lambda n: (n, 2, 0, 0)),
            pl.BlockSpec((1, 1, TC_ROWS, W), lambda n: (n, 3, 0, 0)),
            pl.BlockSpec((1, 1, TC_ROWS, W), lambda n: (n, 4, 0, 0)),
            pl.BlockSpec((1, 1, TC_ROWS, W), lambda n: (n, 2, 0, 0)),
        ],
        out_specs=pl.BlockSpec((1, 4, 8, 128), lambda n: (n, 0, 0, 0)),
        out_shape=jax.ShapeDtypeStruct((N, 4, 8, 128), jnp.int32),
        compiler_params=pltpu.CompilerParams(
            dimension_semantics=("parallel",),
        ),
    )(output, output, output, target)


def _combine_body(sc_ref, tc_ref, o_ref):
    sc = sc_ref[...]
    slo = (sc & 0xFFFF).astype(jnp.float32)
    shi = (sc >> 16).astype(jnp.float32)
    slo = slo.sum(axis=2).reshape(N, 4, 4).sum(axis=1)
    shi = shi.sum(axis=2).reshape(N, 4, 4).sum(axis=1)
    tc = tc_ref[...]
    tlo = (tc & 0xFFFF).astype(jnp.float32).sum(axis=(2, 3))
    thi = (tc >> 16).astype(jnp.float32).sum(axis=(2, 3))
    lo = slo + tlo
    hi = shi + thi
    pi, pi2 = lo[:, 0], hi[:, 0]
    tt, tt2 = lo[:, 1], hi[:, 1]
    i1e, i2e = lo[:, 2], hi[:, 2]
    mq = lo[:, 3]
    m = jnp.float32(H * W)
    p2 = (pi2 - pi) * 0.5
    p1 = pi - 2.0 * p2
    p0 = m - p1 - p2
    t2c = (tt2 - tt) * 0.5
    t1c = tt - 2.0 * t2c
    t0c = m - t1c - t2c
    i2c = (i2e - i1e) * 0.5
    i1c = i1e - 2.0 * i2c
    i0c = mq - i1c - i2c
    eps = jnp.float32(1e-10)
    d0 = jnp.mean(2.0 * i0c / (p0 + t0c + eps))
    d1 = jnp.mean(2.0 * i1c / (p1 + t1c + eps))
    d2 = jnp.mean(2.0 * i2c / (p2 + t2c + eps))
    li = lax.broadcasted_iota(jnp.int32, (8, 128), 1)
    out = jnp.where(li == 0, d0, jnp.where(li == 1, d1,
                    jnp.where(li == 2, d2, 0.0)))
    o_ref[...] = out


def _combine(sc_parts, tc_parts):
    return pl.pallas_call(
        _combine_body,
        out_shape=jax.ShapeDtypeStruct((8, 128), jnp.float32),
    )(sc_parts, tc_parts)


def kernel(output, target):
    tgt = target.astype(jnp.int32)
    sc_parts = _sc_partials(output, tgt)
    tc_parts = _tc_partials(output, tgt)
    res = _combine(sc_parts, tc_parts)
    return res[0, :3]
